# Initial kernel scaffold; baseline (speedup 1.0000x reference)
import jax
import jax.numpy as jnp
from jax.experimental import pallas as pl


def kernel(position_ids, table):
    raise NotImplementedError("write your pallas kernel here")


# speedup vs baseline: 1.9864x; 1.9864x over previous
import functools

import jax
import jax.numpy as jnp
from jax import lax
from jax.experimental import pallas as pl
from jax.experimental.pallas import tpu as pltpu
from jax.experimental.pallas import tpu_sc as plsc

MAX_POS = 8192
D_MODEL = 1024
BATCH = 4
SEQ = 8192

NC = 2
NS = 16
NW = NC * NS

B_TOTAL = BATCH * SEQ
ROWS_PER_W = B_TOTAL // NW
CHUNK = 32
N_CHUNKS = ROWS_PER_W // CHUNK


def _gather_body(table_hbm, ids_hbm, out_hbm, idx_v, rows_v, sem):
    wid = lax.axis_index("s") * NC + lax.axis_index("c")
    pltpu.sync_copy(ids_hbm.at[wid], idx_v)
    row_base = wid * ROWS_PER_W

    def step(c, carry):
        pltpu.async_copy(table_hbm.at[idx_v.at[c]], rows_v, sem).wait()
        pltpu.sync_copy(rows_v, out_hbm.at[pl.ds(row_base + c * CHUNK, CHUNK)])
        return carry

    lax.fori_loop(0, N_CHUNKS, step, 0)


@jax.jit
def _sc_gather(table, ids):
    mesh = plsc.VectorSubcoreMesh(
        core_axis_name="c", subcore_axis_name="s", num_cores=NC, num_subcores=NS
    )
    f = pl.kernel(
        _gather_body,
        out_type=jax.ShapeDtypeStruct((B_TOTAL, D_MODEL), jnp.float32),
        mesh=mesh,
        scratch_types=[
            pltpu.VMEM((N_CHUNKS, CHUNK), jnp.int32),
            pltpu.VMEM((CHUNK, D_MODEL), jnp.float32),
            pltpu.SemaphoreType.DMA,
        ],
    )
    return f(table, ids)


def kernel(position_ids, table):
    ids = position_ids.astype(jnp.int32).reshape(NW, N_CHUNKS, CHUNK)
    out = _sc_gather(table, ids)
    return out.reshape(BATCH, SEQ, D_MODEL)


# speedup vs baseline: 2.2866x; 1.1511x over previous
import functools

import jax
import jax.numpy as jnp
from jax import lax
from jax.experimental import pallas as pl
from jax.experimental.pallas import tpu as pltpu
from jax.experimental.pallas import tpu_sc as plsc

MAX_POS = 8192
D_MODEL = 1024
BATCH = 4
SEQ = 8192

NC = 2
NS = 16
NW = NC * NS

B_TOTAL = BATCH * SEQ
ROWS_PER_W = B_TOTAL // NW
CHUNK = 32
N_CHUNKS = ROWS_PER_W // CHUNK
N_PAIRS = N_CHUNKS // 2


def _gather_body(table_hbm, ids_hbm, out_hbm, idx_v, buf0, buf1, sg0, sg1, so0, so1):
    wid = lax.axis_index("s") * NC + lax.axis_index("c")
    pltpu.sync_copy(ids_hbm.at[wid], idx_v)
    row_base = wid * ROWS_PER_W

    def gather(c, buf, sem):
        return pltpu.make_async_copy(table_hbm.at[idx_v.at[c]], buf, sem)

    def put(c, buf, sem):
        return pltpu.make_async_copy(
            buf, out_hbm.at[pl.ds(row_base + c * CHUNK, CHUNK)], sem
        )

    def pair(g, first, last):
        c0 = 2 * g
        c1 = c0 + 1
        gather(c0, buf0, sg0).wait()
        put(c0, buf0, so0).start()
        if not first:
            put(c0 - 1, buf1, so1).wait()
        gather(c1, buf1, sg1).start()
        gather(c1, buf1, sg1).wait()
        put(c1, buf1, so1).start()
        put(c0, buf0, so0).wait()
        if not last:
            gather(c0 + 2, buf0, sg0).start()

    gather(0, buf0, sg0).start()
    pair(0, first=True, last=False)

    def mid(g, carry):
        pair(g, first=False, last=False)
        return carry

    lax.fori_loop(1, N_PAIRS - 1, mid, 0)
    pair(N_PAIRS - 1, first=False, last=True)
    put(N_CHUNKS - 1, buf1, so1).wait()


@jax.jit
def _sc_gather(table, ids):
    mesh = plsc.VectorSubcoreMesh(
        core_axis_name="c", subcore_axis_name="s", num_cores=NC, num_subcores=NS
    )
    f = pl.kernel(
        _gather_body,
        out_type=jax.ShapeDtypeStruct((B_TOTAL, D_MODEL), jnp.float32),
        mesh=mesh,
        scratch_types=[
            pltpu.VMEM((N_CHUNKS, CHUNK), jnp.int32),
            pltpu.VMEM((CHUNK, D_MODEL), jnp.float32),
            pltpu.VMEM((CHUNK, D_MODEL), jnp.float32),
            pltpu.SemaphoreType.DMA,
            pltpu.SemaphoreType.DMA,
            pltpu.SemaphoreType.DMA,
            pltpu.SemaphoreType.DMA,
        ],
    )
    return f(table, ids)


def kernel(position_ids, table):
    ids = position_ids.astype(jnp.int32).reshape(NW, N_CHUNKS, CHUNK)
    out = _sc_gather(table, ids)
    return out.reshape(BATCH, SEQ, D_MODEL)


# speedup vs baseline: 2.3605x; 1.0323x over previous
import functools

import jax
import jax.numpy as jnp
from jax import lax
from jax.experimental import pallas as pl
from jax.experimental.pallas import tpu as pltpu
from jax.experimental.pallas import tpu_sc as plsc

MAX_POS = 8192
D_MODEL = 1024
BATCH = 4
SEQ = 8192

NC = 2
NS = 16
NW = NC * NS

B_TOTAL = BATCH * SEQ
ROWS_PER_W = B_TOTAL // NW
CHUNK = 16
N_CHUNKS = ROWS_PER_W // CHUNK
NBUF = 4


def _gather_body(table_hbm, ids_hbm, out_hbm, idx_v, bufs, sgs, sos):
    wid = lax.axis_index("s") * NC + lax.axis_index("c")
    pltpu.sync_copy(ids_hbm.at[wid], idx_v)
    row_base = wid * ROWS_PER_W

    def gather(c, p):
        return pltpu.make_async_copy(table_hbm.at[idx_v.at[c]], bufs[p], sgs[p])

    def put(c, p):
        return pltpu.make_async_copy(
            bufs[p], out_hbm.at[pl.ds(row_base + c * CHUNK, CHUNK)], sos[p]
        )

    def step(c, p, first, last):
        gather(c, p).wait()
        put(c, p).start()
        if not first:
            put(c - 2, (p + 2) % NBUF).wait()
        if not last:
            gather(c + 2, (p + 2) % NBUF).start()

    gather(0, 0).start()
    gather(1, 1).start()
    step(0, 0, first=True, last=False)
    step(1, 1, first=True, last=False)

    def group(m, carry):
        c = 4 * m + 2
        for j in range(4):
            step(c + j, (2 + j) % NBUF, first=False, last=False)
        return carry

    lax.fori_loop(0, (N_CHUNKS - 4) // 4, group, 0)
    step(N_CHUNKS - 2, (N_CHUNKS - 2) % NBUF, first=False, last=True)
    step(N_CHUNKS - 1, (N_CHUNKS - 1) % NBUF, first=False, last=True)
    put(N_CHUNKS - 2, (N_CHUNKS - 2) % NBUF).wait()
    put(N_CHUNKS - 1, (N_CHUNKS - 1) % NBUF).wait()


@jax.jit
def _sc_gather(table, ids):
    mesh = plsc.VectorSubcoreMesh(
        core_axis_name="c", subcore_axis_name="s", num_cores=NC, num_subcores=NS
    )
    f = pl.kernel(
        _gather_body,
        out_type=jax.ShapeDtypeStruct((B_TOTAL, D_MODEL), jnp.float32),
        mesh=mesh,
        scratch_types=[
            pltpu.VMEM((N_CHUNKS, CHUNK), jnp.int32),
            [pltpu.VMEM((CHUNK, D_MODEL), jnp.float32) for _ in range(NBUF)],
            [pltpu.SemaphoreType.DMA for _ in range(NBUF)],
            [pltpu.SemaphoreType.DMA for _ in range(NBUF)],
        ],
    )
    return f(table, ids)


def kernel(position_ids, table):
    ids = position_ids.astype(jnp.int32).reshape(NW, N_CHUNKS, CHUNK)
    out = _sc_gather(table, ids)
    return out.reshape(BATCH, SEQ, D_MODEL)
